# Initial kernel scaffold; baseline (speedup 1.0000x reference)
import jax
import jax.numpy as jnp
from jax.experimental import pallas as pl


def kernel(x, edge_index, W1, a_src1, a_dst1, b1, W2, a_src2, a_dst2, b2, W3, a_src3, a_dst3, b3):
    raise NotImplementedError("write your pallas kernel here")


# speedup vs baseline: 25.3755x; 25.3755x over previous
import functools

import jax
import jax.numpy as jnp
from jax import lax
from jax.experimental import pallas as pl
from jax.experimental.pallas import tpu as pltpu
from jax.experimental.pallas import tpu_sc as plsc

N = 10000
E = 320000
IN_DIM = 128
HIDDEN = 256
OUT_DIM = 100
N1 = 10008
NP = 10112
RPT = NP // 16
BATCH = 128
NB = 164
TPB = NB * BATCH
E_PAD = 16 * TPB
E_ROWS = E_PAD // 128
ROWS_PT = TPB // 128


def _mm_body(x_ref, w_ref, a_ref, h_ref, al_ref):
    h = jnp.dot(x_ref[...], w_ref[...], preferred_element_type=jnp.float32)
    h_ref[...] = h
    al_ref[...] = jnp.dot(h, a_ref[...], preferred_element_type=jnp.float32)


def _mm(x, w, a):
    n, din = x.shape
    dout = w.shape[1]
    R = 1000
    return pl.pallas_call(
        _mm_body,
        grid=(n // R,),
        in_specs=[pl.BlockSpec((R, din), lambda i: (i, 0)),
                  pl.BlockSpec((din, dout), lambda i: (0, 0)),
                  pl.BlockSpec((dout, 128), lambda i: (0, 0))],
        out_specs=[pl.BlockSpec((R, dout), lambda i: (i, 0)),
                   pl.BlockSpec((R, 128), lambda i: (i, 0))],
        out_shape=[jax.ShapeDtypeStruct((n, dout), jnp.float32),
                   jax.ShapeDtypeStruct((n, 128), jnp.float32)],
    )(x, w, a)


def _norm_mm_body(acc_ref, den_ref, b_ref, w_ref, a_ref, h_ref, al_ref):
    acc = acc_ref[...]
    d = jnp.sum(den_ref[...], axis=2) + 1e-16
    x = jnp.concatenate([acc[0] / d[0][:, None],
                         acc[1] / d[1][:, None]], axis=1) + b_ref[...]
    x = jnp.where(x > 0.0, x, jnp.exp(x) - 1.0)
    h = jnp.dot(x, w_ref[...], preferred_element_type=jnp.float32)
    h_ref[...] = h
    al_ref[...] = jnp.dot(h, a_ref[...], preferred_element_type=jnp.float32)


def _norm_mm(acc, den_t, b, w, a):
    dout = w.shape[1]
    R = 1000
    return pl.pallas_call(
        _norm_mm_body,
        grid=(N // R,),
        in_specs=[pl.BlockSpec((2, R, 128), lambda i: (0, i, 0)),
                  pl.BlockSpec((2, R, 16), lambda i: (0, i, 0)),
                  pl.BlockSpec((1, HIDDEN), lambda i: (0, 0)),
                  pl.BlockSpec((HIDDEN, dout), lambda i: (0, 0)),
                  pl.BlockSpec((dout, 128), lambda i: (0, 0))],
        out_specs=[pl.BlockSpec((R, dout), lambda i: (i, 0)),
                   pl.BlockSpec((R, 128), lambda i: (i, 0))],
        out_shape=[jax.ShapeDtypeStruct((N, dout), jnp.float32),
                   jax.ShapeDtypeStruct((N, 128), jnp.float32)],
    )(acc, den_t, b.reshape(1, HIDDEN), w, a)


def _final_body(acc_ref, den_ref, b_ref, o_ref):
    acc = acc_ref[...]
    d = jnp.sum(den_ref[...], axis=1) + 1e-16
    o_ref[...] = (acc[0] + acc[1]) / d[:, None] + b_ref[...]


def _final(acc, den_t, b3p):
    R = 1000
    return pl.pallas_call(
        _final_body,
        grid=(N // R,),
        in_specs=[pl.BlockSpec((2, R, 128), lambda i: (0, i, 0)),
                  pl.BlockSpec((R, 32), lambda i: (i, 0)),
                  pl.BlockSpec((1, 128), lambda i: (0, 0))],
        out_specs=pl.BlockSpec((R, 128), lambda i: (i, 0)),
        out_shape=jax.ShapeDtypeStruct((N, 128), jnp.float32),
    )(acc, den_t, b3p)


def _make_edge_sc(core_split):
    hc = 128
    kk = hc // 16
    rows_pt = ROWS_PT // 2 if core_split else ROWS_PT
    nb = NB // 2 if core_split else NB
    mesh = plsc.VectorSubcoreMesh(core_axis_name="c", subcore_axis_name="s")

    def body(src_ref, dst_ref, tab_ref, as_ref, ad_ref,
             acc_out, den_out,
             s_idx, d_idx, p_buf, rows, as_t, ad_t, den_t, acc_sh, gsem):
        c = lax.axis_index("c")
        s = lax.axis_index("s")
        ci = 0 if core_split else c
        pltpu.sync_copy(as_ref.at[ci], as_t)
        pltpu.sync_copy(ad_ref.at[ci], ad_t)
        zv = jnp.zeros((16,), jnp.float32)

        def zden(i, carry):
            den_t[pl.ds(i * 16, 16)] = zv
            return carry
        lax.fori_loop(0, NP // 16, zden, 0)

        def zrow(i, carry):
            for k in range(kk):
                rows[i, pl.ds(k * 16, 16)] = zv
            return carry
        lax.fori_loop(0, BATCH, zrow, 0)

        base = s * RPT

        def zacc(i, carry):
            pltpu.sync_copy(rows, acc_sh.at[pl.ds(base + i * BATCH, BATCH)])
            return carry
        lax.fori_loop(0, RPT // BATCH, zacc, 0)
        pltpu.sync_copy(rows.at[pl.ds(0, RPT % BATCH)],
                        acc_sh.at[pl.ds(base + RPT - RPT % BATCH, RPT % BATCH)])
        plsc.subcore_barrier()

        def batch(g, carry):
            wid = c * 16 + s if core_split else s
            row0 = wid * rows_pt + g
            pltpu.sync_copy(src_ref.at[pl.ds(row0, 1)], s_idx)
            pltpu.sync_copy(dst_ref.at[pl.ds(row0, 1)], d_idx)
            cp = pltpu.make_async_copy(tab_ref.at[ci].at[s_idx.at[0]], rows, gsem)
            cp.start()
            for j in range(8):
                sl = pl.ds(j * 16, 16)
                si = s_idx[0, sl]
                di = d_idx[0, sl]
                e = plsc.load_gather(as_t, [si]) + plsc.load_gather(ad_t, [di])
                e = jnp.maximum(e, e * 0.2)
                p = jnp.exp(e)
                p_buf[sl] = p
                plsc.addupdate_scatter(den_t, [di], p)
            cp.wait()

            def scale(g2, carry2):
                p16 = p_buf[pl.ds(g2 * 16, 16)]
                for l in range(16):
                    pv = p16[l]
                    r = g2 * 16 + l
                    for k in range(kk):
                        sl2 = pl.ds(k * 16, 16)
                        rows[r, sl2] = rows[r, sl2] * pv
                return carry2
            lax.fori_loop(0, BATCH // 16, scale, 0)

            pltpu.sync_copy(rows, acc_sh.at[d_idx.at[0]], add=True)
            return carry
        lax.fori_loop(0, nb, batch, 0)
        plsc.subcore_barrier()

        pltpu.sync_copy(acc_sh.at[pl.ds(base, RPT)],
                        acc_out.at[pl.ds(c * NP + base, RPT)])
        pltpu.sync_copy(den_t, den_out.at[c * 16 + s])

    return pl.kernel(
        body,
        mesh=mesh,
        compiler_params=pltpu.CompilerParams(needs_layout_passes=False),
        out_type=[jax.ShapeDtypeStruct((2 * NP, hc), jnp.float32),
                  jax.ShapeDtypeStruct((32, NP), jnp.float32)],
        scratch_types=[
            pltpu.VMEM((1, 128), jnp.int32),
            pltpu.VMEM((1, 128), jnp.int32),
            pltpu.VMEM((BATCH,), jnp.float32),
            pltpu.VMEM((BATCH, hc), jnp.float32),
            pltpu.VMEM((N1,), jnp.float32),
            pltpu.VMEM((N1,), jnp.float32),
            pltpu.VMEM((NP,), jnp.float32),
            pltpu.VMEM_SHARED((NP, hc), jnp.float32),
            pltpu.SemaphoreType.DMA,
        ],
    )


def _build_A(a_src, a_dst, rows):
    h_cnt, c_dim = a_src.shape
    A = jnp.zeros((rows, 128), jnp.float32)
    for h in range(h_cnt):
        A = A.at[h * c_dim:(h + 1) * c_dim, 2 * h].set(a_src[h])
        A = A.at[h * c_dim:(h + 1) * c_dim, 2 * h + 1].set(a_dst[h])
    return A


def _tables2(h, al):
    z = jnp.zeros((N1 - N, 128), jnp.float32)
    zv = jnp.zeros((N1 - N,), jnp.float32)
    tab = jnp.stack([jnp.concatenate([h[:, :128], z]),
                     jnp.concatenate([h[:, 128:], z])])
    a_s = jnp.stack([jnp.concatenate([al[:, 0], zv]),
                     jnp.concatenate([al[:, 2], zv])])
    a_d = jnp.stack([jnp.concatenate([al[:, 1], zv]),
                     jnp.concatenate([al[:, 3], zv])])
    return tab, a_s, a_d


def _tables3(h, al):
    z = jnp.zeros((N1 - N, 128), jnp.float32)
    zv = jnp.zeros((N1 - N,), jnp.float32)
    tab = jnp.concatenate([h, z])[None]
    a_s1 = jnp.concatenate([al[:, 0], zv])[None]
    a_d1 = jnp.concatenate([al[:, 1], zv])[None]
    return tab, a_s1, a_d1


def kernel(x, edge_index, W1, a_src1, a_dst1, b1,
           W2, a_src2, a_dst2, b2, W3, a_src3, a_dst3, b3):
    ei = edge_index.astype(jnp.int32)
    pad = jnp.full((E_PAD - E - N,), N, jnp.int32)
    loops = jnp.arange(N, dtype=jnp.int32)
    src = jnp.concatenate([ei[0], loops, pad]).reshape(E_ROWS, 128)
    dst = jnp.concatenate([ei[1], loops, pad]).reshape(E_ROWS, 128)

    A1 = _build_A(a_src1, a_dst1, HIDDEN)
    A2 = _build_A(a_src2, a_dst2, HIDDEN)
    A3 = _build_A(a_src3, a_dst3, 128)
    W3p = jnp.concatenate([W3, jnp.zeros((HIDDEN, 128 - OUT_DIM), jnp.float32)], axis=1)
    b3p = jnp.concatenate([b3, jnp.zeros((128 - OUT_DIM,), jnp.float32)]).reshape(1, 128)

    edge2 = _make_edge_sc(False)
    edge3 = _make_edge_sc(True)

    h1, al1 = _mm(x, W1, A1)
    tab1, as1, ad1 = _tables2(h1, al1)
    acc1, den1 = edge2(src, dst, tab1, as1, ad1)
    den1_t = jnp.swapaxes(den1.reshape(2, 16, NP), 1, 2)

    h2, al2 = _norm_mm(acc1.reshape(2, NP, 128), den1_t, b1, W2, A2)
    tab2, as2, ad2 = _tables2(h2, al2)
    acc2, den2 = edge2(src, dst, tab2, as2, ad2)
    den2_t = jnp.swapaxes(den2.reshape(2, 16, NP), 1, 2)

    h3, al3 = _norm_mm(acc2.reshape(2, NP, 128), den2_t, b2, W3p, A3)
    tab3, as3, ad3 = _tables3(h3, al3)
    acc3, den3 = edge3(src, dst, tab3, as3, ad3)
    den3_t = jnp.swapaxes(den3, 0, 1)

    outp = _final(acc3.reshape(2, NP, 128), den3_t, b3p)
    return outp[:, :OUT_DIM]


# speedup vs baseline: 28.7150x; 1.1316x over previous
import functools

import jax
import jax.numpy as jnp
from jax import lax
from jax.experimental import pallas as pl
from jax.experimental.pallas import tpu as pltpu
from jax.experimental.pallas import tpu_sc as plsc

N = 10000
E = 320000
IN_DIM = 128
HIDDEN = 256
OUT_DIM = 100
N1 = 10008
NP = 10112
RPT = NP // 16
BATCH = 128
NB = 164
TPB = NB * BATCH
E_PAD = 16 * TPB
E_ROWS = E_PAD // 128
ROWS_PT = TPB // 128


def _mm_body(x_ref, w_ref, a_ref, h_ref, al_ref):
    h = jnp.dot(x_ref[...], w_ref[...], preferred_element_type=jnp.float32)
    h_ref[...] = h
    al_ref[...] = jnp.dot(h, a_ref[...], preferred_element_type=jnp.float32)


def _mm(x, w, a):
    n, din = x.shape
    dout = w.shape[1]
    R = 1000
    return pl.pallas_call(
        _mm_body,
        grid=(n // R,),
        in_specs=[pl.BlockSpec((R, din), lambda i: (i, 0)),
                  pl.BlockSpec((din, dout), lambda i: (0, 0)),
                  pl.BlockSpec((dout, 128), lambda i: (0, 0))],
        out_specs=[pl.BlockSpec((R, dout), lambda i: (i, 0)),
                   pl.BlockSpec((R, 128), lambda i: (i, 0))],
        out_shape=[jax.ShapeDtypeStruct((n, dout), jnp.float32),
                   jax.ShapeDtypeStruct((n, 128), jnp.float32)],
    )(x, w, a)


def _norm_mm_body(acc_ref, den_ref, b_ref, w_ref, a_ref, h_ref, al_ref):
    acc = acc_ref[...]
    d = jnp.sum(den_ref[...], axis=2) + 1e-16
    x = jnp.concatenate([acc[0] / d[0][:, None],
                         acc[1] / d[1][:, None]], axis=1) + b_ref[...]
    x = jnp.where(x > 0.0, x, jnp.exp(x) - 1.0)
    h = jnp.dot(x, w_ref[...], preferred_element_type=jnp.float32)
    h_ref[...] = h
    al_ref[...] = jnp.dot(h, a_ref[...], preferred_element_type=jnp.float32)


def _norm_mm(acc, den_t, b, w, a):
    dout = w.shape[1]
    R = 1000
    return pl.pallas_call(
        _norm_mm_body,
        grid=(N // R,),
        in_specs=[pl.BlockSpec((2, R, 128), lambda i: (0, i, 0)),
                  pl.BlockSpec((2, R, 16), lambda i: (0, i, 0)),
                  pl.BlockSpec((1, HIDDEN), lambda i: (0, 0)),
                  pl.BlockSpec((HIDDEN, dout), lambda i: (0, 0)),
                  pl.BlockSpec((dout, 128), lambda i: (0, 0))],
        out_specs=[pl.BlockSpec((R, dout), lambda i: (i, 0)),
                   pl.BlockSpec((R, 128), lambda i: (i, 0))],
        out_shape=[jax.ShapeDtypeStruct((N, dout), jnp.float32),
                   jax.ShapeDtypeStruct((N, 128), jnp.float32)],
    )(acc, den_t, b.reshape(1, HIDDEN), w, a)


def _final_body(acc_ref, den_ref, b_ref, o_ref):
    acc = acc_ref[...]
    d = jnp.sum(den_ref[...], axis=1) + 1e-16
    o_ref[...] = (acc[0] + acc[1]) / d[:, None] + b_ref[...]


def _final(acc, den_t, b3p):
    R = 1000
    return pl.pallas_call(
        _final_body,
        grid=(N // R,),
        in_specs=[pl.BlockSpec((2, R, 128), lambda i: (0, i, 0)),
                  pl.BlockSpec((R, 32), lambda i: (i, 0)),
                  pl.BlockSpec((1, 128), lambda i: (0, 0))],
        out_specs=pl.BlockSpec((R, 128), lambda i: (i, 0)),
        out_shape=jax.ShapeDtypeStruct((N, 128), jnp.float32),
    )(acc, den_t, b3p)


def _make_edge_sc(core_split):
    hc = 128
    kk = hc // 16
    rows_pt = ROWS_PT // 2 if core_split else ROWS_PT
    nb = NB // 2 if core_split else NB
    mesh = plsc.VectorSubcoreMesh(core_axis_name="c", subcore_axis_name="s")

    def body(sd_ref, tab_ref, as_ref, ad_ref,
             acc_out, den_out,
             sd_idx, p_buf, rows, as_t, ad_t, den_t, acc_sh, gsem):
        c = lax.axis_index("c")
        s = lax.axis_index("s")
        ci = 0 if core_split else c
        pltpu.sync_copy(as_ref.at[ci], as_t)
        pltpu.sync_copy(ad_ref.at[ci], ad_t)
        zv = jnp.zeros((16,), jnp.float32)

        def zden(i, carry):
            den_t[pl.ds(i * 16, 16)] = zv
            return carry
        lax.fori_loop(0, NP // 16, zden, 0)

        def zrow(i, carry):
            for k in range(kk):
                rows[i, pl.ds(k * 16, 16)] = zv
            return carry
        lax.fori_loop(0, BATCH, zrow, 0)

        base = s * RPT

        def zacc(i, carry):
            pltpu.sync_copy(rows, acc_sh.at[pl.ds(base + i * BATCH, BATCH)])
            return carry
        lax.fori_loop(0, RPT // BATCH, zacc, 0)
        pltpu.sync_copy(rows.at[pl.ds(0, RPT % BATCH)],
                        acc_sh.at[pl.ds(base + RPT - RPT % BATCH, RPT % BATCH)])
        plsc.subcore_barrier()

        def batch(g, carry):
            wid = c * 16 + s if core_split else s
            row0 = wid * rows_pt + g
            pltpu.sync_copy(sd_ref.at[pl.ds(row0, 1)], sd_idx)
            cp = pltpu.make_async_copy(
                tab_ref.at[ci].at[sd_idx.at[0].at[0]], rows, gsem)
            cp.start()
            for j in range(8):
                sl = pl.ds(j * 16, 16)
                si = sd_idx[0, 0, sl]
                di = sd_idx[0, 1, sl]
                e = plsc.load_gather(as_t, [si]) + plsc.load_gather(ad_t, [di])
                e = jnp.maximum(e, e * 0.2)
                p = jnp.exp(e)
                p_buf[sl] = p
                plsc.addupdate_scatter(den_t, [di], p)
            cp.wait()

            def scale(g2, carry2):
                p16 = p_buf[pl.ds(g2 * 16, 16)]
                for l in range(16):
                    pv = p16[l]
                    r = g2 * 16 + l
                    for k in range(kk):
                        sl2 = pl.ds(k * 16, 16)
                        rows[r, sl2] = rows[r, sl2] * pv
                return carry2
            lax.fori_loop(0, BATCH // 16, scale, 0)

            pltpu.sync_copy(rows, acc_sh.at[sd_idx.at[0].at[1]], add=True)
            return carry
        lax.fori_loop(0, nb, batch, 0)
        plsc.subcore_barrier()

        pltpu.sync_copy(acc_sh.at[pl.ds(base, RPT)],
                        acc_out.at[pl.ds(c * NP + base, RPT)])
        pltpu.sync_copy(den_t, den_out.at[c * 16 + s])

    return pl.kernel(
        body,
        mesh=mesh,
        compiler_params=pltpu.CompilerParams(needs_layout_passes=False),
        out_type=[jax.ShapeDtypeStruct((2 * NP, hc), jnp.float32),
                  jax.ShapeDtypeStruct((32, NP), jnp.float32)],
        scratch_types=[
            pltpu.VMEM((1, 2, 128), jnp.int32),
            pltpu.VMEM((BATCH,), jnp.float32),
            pltpu.VMEM((BATCH, hc), jnp.float32),
            pltpu.VMEM((N1,), jnp.float32),
            pltpu.VMEM((N1,), jnp.float32),
            pltpu.VMEM((NP,), jnp.float32),
            pltpu.VMEM_SHARED((NP, hc), jnp.float32),
            pltpu.SemaphoreType.DMA,
        ],
    )


def _build_A(a_src, a_dst, rows):
    h_cnt, c_dim = a_src.shape
    A = jnp.zeros((rows, 128), jnp.float32)
    for h in range(h_cnt):
        A = A.at[h * c_dim:(h + 1) * c_dim, 2 * h].set(a_src[h])
        A = A.at[h * c_dim:(h + 1) * c_dim, 2 * h + 1].set(a_dst[h])
    return A


def _tables2(h, al):
    z = jnp.zeros((N1 - N, 128), jnp.float32)
    zv = jnp.zeros((N1 - N,), jnp.float32)
    tab = jnp.stack([jnp.concatenate([h[:, :128], z]),
                     jnp.concatenate([h[:, 128:], z])])
    a_s = jnp.stack([jnp.concatenate([al[:, 0], zv]),
                     jnp.concatenate([al[:, 2], zv])])
    a_d = jnp.stack([jnp.concatenate([al[:, 1], zv]),
                     jnp.concatenate([al[:, 3], zv])])
    return tab, a_s, a_d


def _tables3(h, al):
    z = jnp.zeros((N1 - N, 128), jnp.float32)
    zv = jnp.zeros((N1 - N,), jnp.float32)
    tab = jnp.concatenate([h, z])[None]
    a_s1 = jnp.concatenate([al[:, 0], zv])[None]
    a_d1 = jnp.concatenate([al[:, 1], zv])[None]
    return tab, a_s1, a_d1


def kernel(x, edge_index, W1, a_src1, a_dst1, b1,
           W2, a_src2, a_dst2, b2, W3, a_src3, a_dst3, b3):
    ei = edge_index.astype(jnp.int32)
    pad = jnp.full((E_PAD - E - N,), N, jnp.int32)
    loops = jnp.arange(N, dtype=jnp.int32)
    src = jnp.concatenate([ei[0], loops, pad]).reshape(E_ROWS, 128)
    dst = jnp.concatenate([ei[1], loops, pad]).reshape(E_ROWS, 128)
    sd = jnp.stack([src, dst], axis=1)

    A1 = _build_A(a_src1, a_dst1, HIDDEN)
    A2 = _build_A(a_src2, a_dst2, HIDDEN)
    A3 = _build_A(a_src3, a_dst3, 128)
    W3p = jnp.concatenate([W3, jnp.zeros((HIDDEN, 128 - OUT_DIM), jnp.float32)], axis=1)
    b3p = jnp.concatenate([b3, jnp.zeros((128 - OUT_DIM,), jnp.float32)]).reshape(1, 128)

    edge2 = _make_edge_sc(False)
    edge3 = _make_edge_sc(True)

    h1, al1 = _mm(x, W1, A1)
    tab1, as1, ad1 = _tables2(h1, al1)
    acc1, den1 = edge2(sd, tab1, as1, ad1)
    den1_t = jnp.swapaxes(den1.reshape(2, 16, NP), 1, 2)

    h2, al2 = _norm_mm(acc1.reshape(2, NP, 128), den1_t, b1, W2, A2)
    tab2, as2, ad2 = _tables2(h2, al2)
    acc2, den2 = edge2(sd, tab2, as2, ad2)
    den2_t = jnp.swapaxes(den2.reshape(2, 16, NP), 1, 2)

    h3, al3 = _norm_mm(acc2.reshape(2, NP, 128), den2_t, b2, W3p, A3)
    tab3, as3, ad3 = _tables3(h3, al3)
    acc3, den3 = edge3(sd, tab3, as3, ad3)
    den3_t = jnp.swapaxes(den3, 0, 1)

    outp = _final(acc3.reshape(2, NP, 128), den3_t, b3p)
    return outp[:, :OUT_DIM]
